# Initial kernel scaffold; baseline (speedup 1.0000x reference)
import jax
import jax.numpy as jnp
from jax.experimental import pallas as pl


def kernel(x_user, x_item, edge_index_ui, edge_index_iu, emb_W_user, emb_b_user, emb_W_item, emb_b_item, l0_ui_Wrel, l0_ui_Wroot, l0_ui_b, l0_iu_Wrel, l0_iu_Wroot, l0_iu_b, l1_ui_Wrel, l1_ui_Wroot, l1_ui_b, l1_iu_Wrel, l1_iu_Wroot, l1_iu_b, out_W_user, out_b_user, out_W_item, out_b_item):
    raise NotImplementedError("write your pallas kernel here")


# speedup vs baseline: 2.8536x; 2.8536x over previous
import jax
import jax.numpy as jnp
from jax import lax
from jax.experimental import pallas as pl
from jax.experimental.pallas import tpu as pltpu
from jax.experimental.pallas import tpu_sc as plsc

N = 10000
H = 128
NSUB = 16
CHUNK = 128
IB = 16

ROWS_PT = ((N + 1 + NSUB * CHUNK - 1) // (NSUB * CHUNK)) * CHUNK
ACC_ROWS = ROWS_PT * NSUB
CROWS = ACC_ROWS // 128
CRPT = 8
NMERGE = CROWS // CRPT


def _mesh():
    return plsc.VectorSubcoreMesh(core_axis_name="c", subcore_axis_name="s")


def _fill(ref, nrows, ncols, val):
    def row(i, carry):
        for j in range(ncols // 16):
            ref[i, pl.ds(j * 16, 16)] = jnp.full((16,), val, ref.dtype)
        return carry
    lax.fori_loop(0, nrows, row, 0)


def _make_agg(nch):
    scratch = [
        pltpu.VMEM_SHARED((ACC_ROWS, H), jnp.float32),
        pltpu.VMEM((IB, CHUNK), jnp.int32),
        pltpu.VMEM((IB, CHUNK), jnp.int32),
        pltpu.VMEM((CHUNK, H), jnp.float32),
        pltpu.SemaphoreType.DMA,
    ]
    assert nch % IB == 0

    def body(y, srcH, dstH, sums_out, acc, src_v, dst_v, rows_v, sem):
        cid = lax.axis_index("c")
        sid = lax.axis_index("s")
        base = sid * ROWS_PT

        _fill(rows_v, CHUNK, H, 0.0)

        def zacc(k, carry):
            pltpu.sync_copy(rows_v, acc.at[pl.ds(base + k * CHUNK, CHUNK)])
            return carry
        lax.fori_loop(0, ROWS_PT // CHUNK, zacc, 0)

        plsc.subcore_barrier()

        def iblock(bk, carry):
            pltpu.sync_copy(srcH.at[cid, sid, pl.ds(bk * IB, IB)], src_v)
            pltpu.sync_copy(dstH.at[cid, sid, pl.ds(bk * IB, IB)], dst_v)

            def chunk(c, carry2):
                pltpu.async_copy(y.at[src_v.at[c]], rows_v, sem).wait()
                pltpu.sync_copy(rows_v, acc.at[dst_v.at[c]], add=True)
                return carry2
            lax.fori_loop(0, IB, chunk, 0)
            return carry
        lax.fori_loop(0, nch // IB, iblock, 0)

        plsc.subcore_barrier()

        pltpu.sync_copy(acc.at[pl.ds(base, ROWS_PT)],
                        sums_out.at[cid, pl.ds(base, ROWS_PT)])

    return pl.kernel(
        body, out_type=jax.ShapeDtypeStruct((2, ACC_ROWS, H), jnp.float32),
        mesh=_mesh(), scratch_types=scratch)


def _make_counts(nch):
    scratch = [
        pltpu.VMEM_SHARED((NSUB, CROWS, 128), jnp.float32),
        pltpu.VMEM((nch, CHUNK), jnp.int32),
        pltpu.VMEM((CROWS, 128), jnp.float32),
        pltpu.VMEM((NSUB, CRPT, 128), jnp.float32),
    ]

    def body(dstH, cnt_out, sh, dst_v, cnt_v, merge_v):
        cid = lax.axis_index("c")
        sid = lax.axis_index("s")
        _fill(cnt_v, CROWS, 128, 0.0)
        pltpu.sync_copy(dstH.at[cid, sid], dst_v)

        ones16 = jnp.full((16,), 1.0, jnp.float32)

        def chunk(c, carry):
            for k in range(CHUNK // 16):
                d = dst_v[c, pl.ds(k * 16, 16)]
                plsc.addupdate_scatter(
                    cnt_v, [lax.shift_right_logical(d, 7),
                            lax.bitwise_and(d, 127)], ones16)
            return carry
        lax.fori_loop(0, nch, chunk, 0)

        pltpu.sync_copy(cnt_v, sh.at[sid])
        plsc.subcore_barrier()

        @pl.when(sid < NMERGE)
        def _():
            pltpu.sync_copy(sh.at[:, pl.ds(sid * CRPT, CRPT)], merge_v)
            for r in range(CRPT):
                for j in range(128 // 16):
                    sl = pl.ds(j * 16, 16)
                    tot = merge_v[0, r, sl]
                    for t in range(1, NSUB):
                        tot = tot + merge_v[t, r, sl]
                    cnt_v[r, sl] = tot
            pltpu.sync_copy(cnt_v.at[pl.ds(0, CRPT)],
                            cnt_out.at[cid, pl.ds(sid * CRPT, CRPT)])

    return pl.kernel(
        body, out_type=jax.ShapeDtypeStruct((2, CROWS, 128), jnp.float32),
        mesh=_mesh(), scratch_types=scratch,
        compiler_params=pltpu.CompilerParams(needs_layout_passes=False))


_B = 2000
_NB = N // _B


def _dot(a, b):
    return jnp.dot(a, b, preferred_element_type=jnp.float32,
                   precision=lax.Precision.HIGHEST)


def _prologue_body(x, We, be, Wrel, Wroot, brel, y, root):
    h = jnp.maximum(_dot(x[...], We[0]) + be[0], 0.0)
    y[...] = _dot(h, Wrel[0])
    root[...] = _dot(h, Wroot[0]) + brel[0]


def _mid_body(s, c, r0, Wrel, Wroot, b, y, r1):
    cnt = jnp.maximum(c[0], 1.0)
    h = jnp.maximum(s[0] / cnt + r0[...], 0.0)
    y[...] = _dot(h, Wrel[0])
    r1[...] = _dot(h, Wroot[0]) + b[0]


def _epi_body(s, c, r1, Wout, bout, o):
    cnt = jnp.maximum(c[0], 1.0)
    agg = s[0] / cnt + r1[...]
    o[...] = _dot(agg, Wout[0]) + bout[0]


def _row_spec(g_cross=False):
    f = (lambda g, b: ((1 - g) * _NB + b, 0)) if g_cross else \
        (lambda g, b: (g * _NB + b, 0))
    return pl.BlockSpec((_B, H), f)


def _acc_spec(width, g_cross=False):
    f = (lambda g, b: (1 - g, b, 0)) if g_cross else (lambda g, b: (g, b, 0))
    return pl.BlockSpec((1, _B, width), f)


def _w_spec(shape):
    return pl.BlockSpec((1,) + shape, lambda g, b: (g,) + (0,) * len(shape))


_prologue = pl.pallas_call(
    _prologue_body,
    grid=(2, _NB),
    in_specs=[
        pl.BlockSpec((_B, 3), lambda g, b: (g * _NB + b, 0)),
        _w_spec((3, H)), _w_spec((1, H)),
        _w_spec((H, H)), _w_spec((H, H)), _w_spec((1, H)),
    ],
    out_specs=[_row_spec(), _row_spec()],
    out_shape=[jax.ShapeDtypeStruct((2 * N, H), jnp.float32),
               jax.ShapeDtypeStruct((2 * N, H), jnp.float32)],
)

_mid = pl.pallas_call(
    _mid_body,
    grid=(2, _NB),
    in_specs=[
        _acc_spec(H, g_cross=True),
        _acc_spec(1, g_cross=True),
        _row_spec(g_cross=True),
        _w_spec((H, H)), _w_spec((H, H)), _w_spec((1, H)),
    ],
    out_specs=[_row_spec(), _row_spec()],
    out_shape=[jax.ShapeDtypeStruct((2 * N, H), jnp.float32),
               jax.ShapeDtypeStruct((2 * N, H), jnp.float32)],
)

_epi = pl.pallas_call(
    _epi_body,
    grid=(2, _NB),
    in_specs=[
        _acc_spec(H),
        _acc_spec(1),
        _row_spec(),
        _w_spec((H, H)), _w_spec((1, H)),
    ],
    out_specs=_row_spec(),
    out_shape=jax.ShapeDtypeStruct((2 * N, H), jnp.float32),
)


def kernel(x_user, x_item, edge_index_ui, edge_index_iu,
           emb_W_user, emb_b_user, emb_W_item, emb_b_item,
           l0_ui_Wrel, l0_ui_Wroot, l0_ui_b,
           l0_iu_Wrel, l0_iu_Wroot, l0_iu_b,
           l1_ui_Wrel, l1_ui_Wroot, l1_ui_b,
           l1_iu_Wrel, l1_iu_Wroot, l1_iu_b,
           out_W_user, out_b_user, out_W_item, out_b_item):
    E = edge_index_iu.shape[1]
    nch = -(-E // (NSUB * CHUNK))
    nch = -(-nch // IB) * IB
    epad = NSUB * nch * CHUNK - E

    def prep(ei, off):
        src = jnp.concatenate([ei[0] + off,
                               jnp.zeros((epad,), jnp.int32)])
        dst = jnp.concatenate([ei[1], jnp.full((epad,), N, jnp.int32)])
        return (src.reshape(NSUB, nch, CHUNK),
                dst.reshape(NSUB, nch, CHUNK))

    s0, d0 = prep(edge_index_iu, 0)
    s1, d1 = prep(edge_index_ui, N)
    srcH = jnp.stack([s0, s1])
    dstH = jnp.stack([d0, d1])

    x_cat = jnp.concatenate([x_item, x_user], axis=0)
    We = jnp.stack([emb_W_item, emb_W_user])
    be = jnp.stack([emb_b_item, emb_b_user])[:, None, :]
    Wrel0 = jnp.stack([l0_iu_Wrel, l0_ui_Wrel])
    Wroot0 = jnp.stack([l0_iu_Wroot, l0_ui_Wroot])
    b0 = jnp.stack([l0_iu_b, l0_ui_b])[:, None, :]
    Wrel1 = jnp.stack([l1_iu_Wrel, l1_ui_Wrel])
    Wroot1 = jnp.stack([l1_iu_Wroot, l1_ui_Wroot])
    b1 = jnp.stack([l1_iu_b, l1_ui_b])[:, None, :]
    C = out_W_user.shape[1]
    Wout = jnp.zeros((2, H, H), jnp.float32)
    Wout = Wout.at[0, :, :C].set(out_W_user).at[1, :, :C].set(out_W_item)
    bout = jnp.zeros((2, 1, H), jnp.float32)
    bout = bout.at[0, 0, :C].set(out_b_user).at[1, 0, :C].set(out_b_item)

    y0, root0 = _prologue(x_cat, We, be, Wrel0, Wroot0, b0)
    cnts = _make_counts(nch)(dstH).reshape(2, ACC_ROWS, 1)
    sums0 = _make_agg(nch)(y0, srcH, dstH)
    y1, root1 = _mid(sums0, cnts, root0, Wrel1, Wroot1, b1)
    sums1 = _make_agg(nch)(y1, srcH, dstH)
    o = _epi(sums1, cnts, root1, Wout, bout)
    return o[:N, :C], o[N:, :C]


# speedup vs baseline: 3.0631x; 1.0734x over previous
import jax
import jax.numpy as jnp
from jax import lax
from jax.experimental import pallas as pl
from jax.experimental.pallas import tpu as pltpu
from jax.experimental.pallas import tpu_sc as plsc

N = 10000
H = 128
NSUB = 16
CHUNK = 128
IB = 32

ROWS_PT = ((N + 1 + NSUB * CHUNK - 1) // (NSUB * CHUNK)) * CHUNK
ACC_ROWS = ROWS_PT * NSUB
CROWS = ACC_ROWS // 128
CRPT = 8
NMERGE = CROWS // CRPT


def _mesh():
    return plsc.VectorSubcoreMesh(core_axis_name="c", subcore_axis_name="s")


def _fill(ref, nrows, ncols, val):
    def row(i, carry):
        for j in range(ncols // 16):
            ref[i, pl.ds(j * 16, 16)] = jnp.full((16,), val, ref.dtype)
        return carry
    lax.fori_loop(0, nrows, row, 0)


def _make_agg(nch):
    scratch = [
        pltpu.VMEM_SHARED((ACC_ROWS, H), jnp.float32),
        pltpu.VMEM((IB, CHUNK), jnp.int32),
        pltpu.VMEM((IB, CHUNK), jnp.int32),
        pltpu.VMEM((2, CHUNK, H), jnp.float32),
        pltpu.SemaphoreType.DMA((2,)),
        pltpu.SemaphoreType.DMA((2,)),
    ]
    assert nch % IB == 0 and IB % 2 == 0

    def body(y, srcH, dstH, sums_out, acc, src_v, dst_v, rows, gsem, ssem):
        cid = lax.axis_index("c")
        sid = lax.axis_index("s")
        base = sid * ROWS_PT

        def start_g(c, b):
            pltpu.async_copy(y.at[src_v.at[c]], rows.at[b], gsem.at[b])

        def wait_g(c, b):
            pltpu.make_async_copy(y.at[src_v.at[c]], rows.at[b],
                                  gsem.at[b]).wait()

        def start_s(c, b):
            pltpu.async_copy(rows.at[b], acc.at[dst_v.at[c]], ssem.at[b],
                             add=True)

        def wait_s(c, b):
            pltpu.make_async_copy(rows.at[b], acc.at[dst_v.at[c]],
                                  ssem.at[b]).wait()

        _fill(rows.at[0], CHUNK, H, 0.0)

        def zacc(k, carry):
            pltpu.sync_copy(rows.at[0], acc.at[pl.ds(base + k * CHUNK, CHUNK)])
            return carry
        lax.fori_loop(0, ROWS_PT // CHUNK, zacc, 0)

        plsc.subcore_barrier()

        def iblock(bk, carry):
            pltpu.sync_copy(srcH.at[cid, sid, pl.ds(bk * IB, IB)], src_v)
            pltpu.sync_copy(dstH.at[cid, sid, pl.ds(bk * IB, IB)], dst_v)
            start_g(0, 0)

            def pair(k, carry2):
                c0 = 2 * k
                wait_g(c0, 0)
                start_s(c0, 0)

                @pl.when(k > 0)
                def _():
                    wait_s(c0 - 1, 1)
                start_g(c0 + 1, 1)
                wait_g(c0 + 1, 1)
                start_s(c0 + 1, 1)
                wait_s(c0, 0)

                @pl.when(k < IB // 2 - 1)
                def _():
                    start_g(c0 + 2, 0)
                return carry2
            lax.fori_loop(0, IB // 2, pair, 0)
            wait_s(IB - 1, 1)
            return carry
        lax.fori_loop(0, nch // IB, iblock, 0)

        plsc.subcore_barrier()

        pltpu.sync_copy(acc.at[pl.ds(base, ROWS_PT)],
                        sums_out.at[cid, pl.ds(base, ROWS_PT)])

    return pl.kernel(
        body, out_type=jax.ShapeDtypeStruct((2, ACC_ROWS, H), jnp.float32),
        mesh=_mesh(), scratch_types=scratch)


def _make_counts(nch):
    scratch = [
        pltpu.VMEM_SHARED((NSUB, CROWS, 128), jnp.float32),
        pltpu.VMEM((nch, CHUNK), jnp.int32),
        pltpu.VMEM((CROWS, 128), jnp.float32),
        pltpu.VMEM((NSUB, CRPT, 128), jnp.float32),
    ]

    def body(dstH, cnt_out, sh, dst_v, cnt_v, merge_v):
        cid = lax.axis_index("c")
        sid = lax.axis_index("s")
        _fill(cnt_v, CROWS, 128, 0.0)
        pltpu.sync_copy(dstH.at[cid, sid], dst_v)

        ones16 = jnp.full((16,), 1.0, jnp.float32)

        def chunk(c, carry):
            for k in range(CHUNK // 16):
                d = dst_v[c, pl.ds(k * 16, 16)]
                plsc.addupdate_scatter(
                    cnt_v, [lax.shift_right_logical(d, 7),
                            lax.bitwise_and(d, 127)], ones16)
            return carry
        lax.fori_loop(0, nch, chunk, 0)

        pltpu.sync_copy(cnt_v, sh.at[sid])
        plsc.subcore_barrier()

        @pl.when(sid < NMERGE)
        def _():
            pltpu.sync_copy(sh.at[:, pl.ds(sid * CRPT, CRPT)], merge_v)
            for r in range(CRPT):
                for j in range(128 // 16):
                    sl = pl.ds(j * 16, 16)
                    tot = merge_v[0, r, sl]
                    for t in range(1, NSUB):
                        tot = tot + merge_v[t, r, sl]
                    cnt_v[r, sl] = tot
            pltpu.sync_copy(cnt_v.at[pl.ds(0, CRPT)],
                            cnt_out.at[cid, pl.ds(sid * CRPT, CRPT)])

    return pl.kernel(
        body, out_type=jax.ShapeDtypeStruct((2, CROWS, 128), jnp.float32),
        mesh=_mesh(), scratch_types=scratch,
        compiler_params=pltpu.CompilerParams(needs_layout_passes=False))


_B = 2000
_NB = N // _B


def _dot(a, b):
    return jnp.dot(a, b, preferred_element_type=jnp.float32,
                   precision=lax.Precision.HIGHEST)


def _prologue_body(x, We, be, Wrel, Wroot, brel, y, root):
    h = jnp.maximum(_dot(x[...], We[0]) + be[0], 0.0)
    y[...] = _dot(h, Wrel[0])
    root[...] = _dot(h, Wroot[0]) + brel[0]


def _mid_body(s, c, r0, Wrel, Wroot, b, y, r1):
    cnt = jnp.maximum(c[0], 1.0)
    h = jnp.maximum(s[0] / cnt + r0[...], 0.0)
    y[...] = _dot(h, Wrel[0])
    r1[...] = _dot(h, Wroot[0]) + b[0]


def _epi_body(s, c, r1, Wout, bout, o):
    cnt = jnp.maximum(c[0], 1.0)
    agg = s[0] / cnt + r1[...]
    o[...] = _dot(agg, Wout[0]) + bout[0]


def _row_spec(g_cross=False):
    f = (lambda g, b: ((1 - g) * _NB + b, 0)) if g_cross else \
        (lambda g, b: (g * _NB + b, 0))
    return pl.BlockSpec((_B, H), f)


def _acc_spec(width, g_cross=False):
    f = (lambda g, b: (1 - g, b, 0)) if g_cross else (lambda g, b: (g, b, 0))
    return pl.BlockSpec((1, _B, width), f)


def _w_spec(shape):
    return pl.BlockSpec((1,) + shape, lambda g, b: (g,) + (0,) * len(shape))


_prologue = pl.pallas_call(
    _prologue_body,
    grid=(2, _NB),
    in_specs=[
        pl.BlockSpec((_B, 3), lambda g, b: (g * _NB + b, 0)),
        _w_spec((3, H)), _w_spec((1, H)),
        _w_spec((H, H)), _w_spec((H, H)), _w_spec((1, H)),
    ],
    out_specs=[_row_spec(), _row_spec()],
    out_shape=[jax.ShapeDtypeStruct((2 * N, H), jnp.float32),
               jax.ShapeDtypeStruct((2 * N, H), jnp.float32)],
)

_mid = pl.pallas_call(
    _mid_body,
    grid=(2, _NB),
    in_specs=[
        _acc_spec(H, g_cross=True),
        _acc_spec(1, g_cross=True),
        _row_spec(g_cross=True),
        _w_spec((H, H)), _w_spec((H, H)), _w_spec((1, H)),
    ],
    out_specs=[_row_spec(), _row_spec()],
    out_shape=[jax.ShapeDtypeStruct((2 * N, H), jnp.float32),
               jax.ShapeDtypeStruct((2 * N, H), jnp.float32)],
)

_epi = pl.pallas_call(
    _epi_body,
    grid=(2, _NB),
    in_specs=[
        _acc_spec(H),
        _acc_spec(1),
        _row_spec(),
        _w_spec((H, H)), _w_spec((1, H)),
    ],
    out_specs=_row_spec(),
    out_shape=jax.ShapeDtypeStruct((2 * N, H), jnp.float32),
)


def kernel(x_user, x_item, edge_index_ui, edge_index_iu,
           emb_W_user, emb_b_user, emb_W_item, emb_b_item,
           l0_ui_Wrel, l0_ui_Wroot, l0_ui_b,
           l0_iu_Wrel, l0_iu_Wroot, l0_iu_b,
           l1_ui_Wrel, l1_ui_Wroot, l1_ui_b,
           l1_iu_Wrel, l1_iu_Wroot, l1_iu_b,
           out_W_user, out_b_user, out_W_item, out_b_item):
    E = edge_index_iu.shape[1]
    nch = -(-E // (NSUB * CHUNK))
    nch = -(-nch // IB) * IB
    epad = NSUB * nch * CHUNK - E

    def prep(ei, off):
        src = jnp.concatenate([ei[0] + off,
                               jnp.zeros((epad,), jnp.int32)])
        dst = jnp.concatenate([ei[1], jnp.full((epad,), N, jnp.int32)])
        return (src.reshape(NSUB, nch, CHUNK),
                dst.reshape(NSUB, nch, CHUNK))

    s0, d0 = prep(edge_index_iu, 0)
    s1, d1 = prep(edge_index_ui, N)
    srcH = jnp.stack([s0, s1])
    dstH = jnp.stack([d0, d1])

    x_cat = jnp.concatenate([x_item, x_user], axis=0)
    We = jnp.stack([emb_W_item, emb_W_user])
    be = jnp.stack([emb_b_item, emb_b_user])[:, None, :]
    Wrel0 = jnp.stack([l0_iu_Wrel, l0_ui_Wrel])
    Wroot0 = jnp.stack([l0_iu_Wroot, l0_ui_Wroot])
    b0 = jnp.stack([l0_iu_b, l0_ui_b])[:, None, :]
    Wrel1 = jnp.stack([l1_iu_Wrel, l1_ui_Wrel])
    Wroot1 = jnp.stack([l1_iu_Wroot, l1_ui_Wroot])
    b1 = jnp.stack([l1_iu_b, l1_ui_b])[:, None, :]
    C = out_W_user.shape[1]
    Wout = jnp.zeros((2, H, H), jnp.float32)
    Wout = Wout.at[0, :, :C].set(out_W_user).at[1, :, :C].set(out_W_item)
    bout = jnp.zeros((2, 1, H), jnp.float32)
    bout = bout.at[0, 0, :C].set(out_b_user).at[1, 0, :C].set(out_b_item)

    y0, root0 = _prologue(x_cat, We, be, Wrel0, Wroot0, b0)
    cnts = _make_counts(nch)(dstH).reshape(2, ACC_ROWS, 1)
    sums0 = _make_agg(nch)(y0, srcH, dstH)
    y1, root1 = _mid(sums0, cnts, root0, Wrel1, Wroot1, b1)
    sums1 = _make_agg(nch)(y1, srcH, dstH)
    o = _epi(sums1, cnts, root1, Wout, bout)
    return o[:N, :C], o[N:, :C]
